# baseline (speedup 1.0000x reference)
import jax
import jax.numpy as jnp
from jax import lax
from jax.experimental import pallas as pl
from jax.experimental.pallas import tpu as pltpu
from jax.experimental.pallas import tpu_sc as plsc

_B, _L, _H = 1024, 200, 64
_NC, _NS = 2, 16
_NW = _NC * _NS
_CH = 128
_TOK = _B * _L
_GCH = _TOK // _CH
_NCH = _GCH // _NW


def _emb_body(ids_hbm, tok_hbm, pos_hbm, out_hbm, idx_v, pos_v, buf_v, gsem, osem):
    wid = lax.axis_index("s") * _NC + lax.axis_index("c")
    base = wid * _NCH
    pltpu.sync_copy(ids_hbm.at[pl.ds(base, _NCH)], idx_v)
    pltpu.sync_copy(pos_hbm.at[pl.ds(0, _L)], pos_v.at[pl.ds(0, _L)])
    pltpu.sync_copy(pos_hbm.at[pl.ds(0, _L)], pos_v.at[pl.ds(_L, _L)])

    pltpu.async_copy(tok_hbm.at[idx_v.at[0]], buf_v.at[0], gsem)

    def chunk_body(c, carry):
        s = c % 2

        @pl.when(c >= 1)
        def _():
            pltpu.make_async_copy(buf_v.at[1 - s], out_hbm.at[base + c - 1], osem).wait()

        @pl.when(c + 1 < _NCH)
        def _():
            pltpu.async_copy(tok_hbm.at[idx_v.at[c + 1]], buf_v.at[1 - s], gsem)

        pltpu.make_async_copy(tok_hbm.at[idx_v.at[c]], buf_v.at[s], gsem).wait()

        base_p = (c * _CH) % _L

        def row_body(i, carry2):
            t = i * 2
            for u in range(2):
                p = base_p + t + u
                for d in range(_H // 16):
                    sl = pl.ds(d * 16, 16)
                    buf_v[s, t + u, sl] = buf_v[s, t + u, sl] + pos_v[p, sl]
            return carry2

        lax.fori_loop(0, _CH // 2, row_body, 0)

        pltpu.async_copy(buf_v.at[s], out_hbm.at[base + c], osem)
        return carry

    lax.fori_loop(0, _NCH, chunk_body, 0)
    pltpu.make_async_copy(buf_v.at[(_NCH - 1) % 2], out_hbm.at[base + _NCH - 1], osem).wait()


@jax.jit
def kernel(input_ids, token_table, pos_table):
    ids = input_ids.reshape(_GCH, _CH).astype(jnp.int32)
    out = pl.kernel(
        _emb_body,
        out_type=jax.ShapeDtypeStruct((_GCH, _CH, _H), jnp.float32),
        mesh=plsc.VectorSubcoreMesh(core_axis_name="c", subcore_axis_name="s"),
        compiler_params=pltpu.CompilerParams(use_tc_tiling_on_sc=False),
        scratch_types=[
            pltpu.VMEM((_NCH, _CH), jnp.int32),
            pltpu.VMEM((2 * _L, _H), jnp.float32),
            pltpu.VMEM((2, _CH, _H), jnp.float32),
            pltpu.SemaphoreType.DMA,
            pltpu.SemaphoreType.DMA,
        ],
    )(ids, token_table, pos_table)
    return out.reshape(_B, _L, _H)


# speedup vs baseline: 1.1412x; 1.1412x over previous
import jax
import jax.numpy as jnp
from jax import lax
from jax.experimental import pallas as pl
from jax.experimental.pallas import tpu as pltpu
from jax.experimental.pallas import tpu_sc as plsc

_B, _L, _H = 1024, 200, 64
_NC, _NS = 2, 16
_NW = _NC * _NS
_CH = 100
_TOK = _B * _L
_GCH = _TOK // _CH
_NCH = _GCH // _NW


def _emb_body(ids_hbm, tok_hbm, pos_hbm, out_hbm, idx_v, pos_sh, buf_v, gsem, osem):
    sid = lax.axis_index("s")
    wid = sid * _NC + lax.axis_index("c")
    base = wid * _NCH
    pltpu.sync_copy(ids_hbm.at[pl.ds(base, _NCH)], idx_v)

    @pl.when(sid == 0)
    def _():
        pltpu.sync_copy(pos_hbm.at[pl.ds(0, _L)], pos_sh)

    plsc.subcore_barrier()

    pltpu.sync_copy(pos_sh.at[pl.ds(0, _CH)], buf_v.at[0])
    pltpu.async_copy(tok_hbm.at[idx_v.at[0]], buf_v.at[0], gsem, add=True)

    def chunk_body(c, carry):
        s = c % 2

        @pl.when(c >= 1)
        def _():
            pltpu.make_async_copy(buf_v.at[1 - s], out_hbm.at[base + c - 1], osem).wait()

        @pl.when(c + 1 < _NCH)
        def _():
            pltpu.sync_copy(pos_sh.at[pl.ds(((c + 1) % 2) * _CH, _CH)], buf_v.at[1 - s])
            pltpu.async_copy(tok_hbm.at[idx_v.at[c + 1]], buf_v.at[1 - s], gsem, add=True)

        pltpu.make_async_copy(tok_hbm.at[idx_v.at[c]], buf_v.at[s], gsem).wait()

        pltpu.async_copy(buf_v.at[s], out_hbm.at[base + c], osem)
        return carry

    lax.fori_loop(0, _NCH, chunk_body, 0)
    pltpu.make_async_copy(buf_v.at[1], out_hbm.at[base + _NCH - 1], osem).wait()


@jax.jit
def kernel(input_ids, token_table, pos_table):
    ids = input_ids.reshape(_GCH, _CH).astype(jnp.int32)
    out = pl.kernel(
        _emb_body,
        out_type=jax.ShapeDtypeStruct((_GCH, _CH, _H), jnp.float32),
        mesh=plsc.VectorSubcoreMesh(core_axis_name="c", subcore_axis_name="s"),
        compiler_params=pltpu.CompilerParams(use_tc_tiling_on_sc=False),
        scratch_types=[
            pltpu.VMEM((_NCH, _CH), jnp.int32),
            pltpu.VMEM_SHARED((_L, _H), jnp.float32),
            pltpu.VMEM((2, _CH, _H), jnp.float32),
            pltpu.SemaphoreType.DMA,
            pltpu.SemaphoreType.DMA,
        ],
    )(ids, token_table, pos_table)
    return out.reshape(_B, _L, _H)


# speedup vs baseline: 1.1647x; 1.0205x over previous
import jax
import jax.numpy as jnp
from jax import lax
from jax.experimental import pallas as pl
from jax.experimental.pallas import tpu as pltpu
from jax.experimental.pallas import tpu_sc as plsc

_B, _L, _H = 1024, 200, 64
_NC, _NS = 2, 16
_NW = _NC * _NS
_CH = 64
_TOK = _B * _L
_GCH = _TOK // _CH
_NCH = _GCH // _NW


def _emb_body(ids_hbm, tok_hbm, pos_hbm, out_hbm,
              idx_v, pos_sh, buf_v, obuf_v, gsem, osem):
    sid = lax.axis_index("s")
    wid = sid * _NC + lax.axis_index("c")
    base = wid * _NCH
    pltpu.sync_copy(ids_hbm.at[pl.ds(base, _NCH)], idx_v)

    @pl.when(sid == 0)
    def _():
        pltpu.sync_copy(pos_hbm.at[pl.ds(0, _L)], pos_sh.at[pl.ds(0, _L)])
        pltpu.sync_copy(pos_hbm.at[pl.ds(0, _L)], pos_sh.at[pl.ds(_L, _L)])

    plsc.subcore_barrier()

    def prefill_and_gather(c, slot):
        base_p = (c * _CH) % _L
        pltpu.sync_copy(pos_sh.at[pl.ds(base_p, _CH)], buf_v.at[slot])
        pltpu.async_copy(tok_hbm.at[idx_v.at[c]], buf_v.at[slot], gsem, add=True)

    prefill_and_gather(0, 0)

    def chunk_body(c, carry):
        s = c % 2

        @pl.when(c >= 2)
        def _():
            pltpu.make_async_copy(obuf_v.at[s], out_hbm.at[base + c - 2], osem).wait()

        @pl.when(c + 1 < _NCH)
        def _():
            prefill_and_gather(c + 1, 1 - s)

        pltpu.make_async_copy(tok_hbm.at[idx_v.at[c]], buf_v.at[s], gsem).wait()

        def row_body(t, carry2):
            for d in range(_H // 16):
                sl = pl.ds(d * 16, 16)
                obuf_v[s, t, sl] = buf_v[s, t, sl]
            return carry2

        lax.fori_loop(0, _CH, row_body, 0)

        pltpu.async_copy(obuf_v.at[s], out_hbm.at[base + c], osem)
        return carry

    lax.fori_loop(0, _NCH, chunk_body, 0)
    pltpu.make_async_copy(obuf_v.at[_NCH % 2], out_hbm.at[base + _NCH - 2], osem).wait()
    pltpu.make_async_copy(obuf_v.at[(_NCH - 1) % 2], out_hbm.at[base + _NCH - 1], osem).wait()


@jax.jit
def kernel(input_ids, token_table, pos_table):
    ids = input_ids.reshape(_GCH, _CH).astype(jnp.int32)
    out = pl.kernel(
        _emb_body,
        out_type=jax.ShapeDtypeStruct((_GCH, _CH, 2 * _H), jnp.float32),
        mesh=plsc.VectorSubcoreMesh(core_axis_name="c", subcore_axis_name="s"),
        compiler_params=pltpu.CompilerParams(use_tc_tiling_on_sc=False),
        scratch_types=[
            pltpu.VMEM((_NCH, _CH), jnp.int32),
            pltpu.VMEM_SHARED((2 * _L, _H), jnp.float32),
            pltpu.VMEM((2, _CH, _H), jnp.float32),
            pltpu.VMEM((2, _CH, 2 * _H), jnp.float32),
            pltpu.SemaphoreType.DMA,
            pltpu.SemaphoreType.DMA,
        ],
    )(ids, token_table, pos_table)
    return out[:, :, :_H].reshape(_B, _L, _H)
